# baseline (speedup 1.0000x reference)
import jax
import jax.numpy as jnp
from jax import lax
from jax.experimental import pallas as pl
from jax.experimental.pallas import tpu as pltpu
from jax.experimental.pallas import tpu_sc as plsc

NUM_USER = 30000
NUM_GROUP = 20000
N = NUM_USER + NUM_GROUP
E = 800000
D = 64
HD = D // 2
LAYERS = 3
B = 4096

NC = 2
NS = 16
CHUNK = 128
SLOT = 256
SETS = 3
TILE_EDGES = E // NS
NSLOTS_TILE = 201
TILE_SPAN = NSLOTS_TILE * SLOT
E_PAD = NS * TILE_SPAN + 2 * SLOT
RCHUNK = 40
NRCHUNKS = N // RCHUNK
EGO_CHUNK = 16
BCHUNK = 128


def _sc_body(all0_cs, user_table, group_table, adj_valp, adj_rowp2, adj_colp2,
             user_inputs, pos_groups, neg_groups,
             u_cs, p_cs, n_cs, u_ego, p_ego, n_ego, layers_hbm,
             col0, col1, col2, row0, row1, row2, srow0, srow1, srow2,
             val0, val1, val2, rows0, rows1, rows2,
             tmpv, ebidx, egorows, acc_sh,
             semI0, semI1, semI2, semG0, semG1, semG2,
             semS0, semS1, semS2, gsem):
    c = lax.axis_index("c")
    s = lax.axis_index("s")

    cols = [col0, col1, col2]
    rowsb = [row0, row1, row2]
    srows = [srow0, srow1, srow2]
    vals = [val0, val1, val2]
    rowsd = [rows0, rows1, rows2]
    semI = [semI0, semI1, semI2]
    semG = [semG0, semG1, semG2]
    semS = [semS0, semS1, semS2]

    nrchunks_tile = (NRCHUNKS - s + NS - 1) // NS
    base_e = s * TILE_SPAN
    base_ch = s * (TILE_SPAN // CHUNK)

    def issue_idx(j, X):
        e0 = base_e + j * SLOT
        ch0 = base_ch + 2 * j
        pltpu.async_copy(adj_colp2.at[pl.ds(ch0, 2)], cols[X], semI[X])
        pltpu.async_copy(adj_rowp2.at[pl.ds(ch0, 2)], rowsb[X], semI[X])
        pltpu.async_copy(adj_valp.at[pl.ds(e0, SLOT)], vals[X], semI[X])

    def drain_idx(X):
        pltpu.make_async_copy(adj_colp2.at[pl.ds(0, 2)], cols[X], semI[X]).wait()
        pltpu.make_async_copy(adj_rowp2.at[pl.ds(0, 2)], rowsb[X], semI[X]).wait()
        pltpu.make_async_copy(adj_valp.at[pl.ds(0, SLOT)], vals[X], semI[X]).wait()

    def snap_srow(X):
        for h in range(2):
            for ii in range(8):
                sl = pl.ds(ii * 16, 16)
                srows[X][h, sl] = rowsb[X][h, sl]

    def mul_slot(X):
        def mbody(q, _):
            for hh in range(4):
                g0 = q * 4 + hh
                v0 = g0 * 16
                vv = vals[X][pl.ds(v0, 16)]
                h = g0 // 8
                for j in range(16):
                    vj = vv[j]
                    er = v0 + j - h * 128
                    rowsd[X][h, er, pl.ds(0, 16)] = \
                        rowsd[X][h, er, pl.ds(0, 16)] * vj
                    rowsd[X][h, er, pl.ds(16, 16)] = \
                        rowsd[X][h, er, pl.ds(16, 16)] * vj
            return 0
        lax.fori_loop(0, 4, mbody, 0)

    def zero_set(X):
        def zr(r, _):
            for h in range(2):
                rowsd[X][h, r, pl.ds(0, 16)] = jnp.zeros((16,), jnp.float32)
                rowsd[X][h, r, pl.ds(16, 16)] = jnp.zeros((16,), jnp.float32)
            return 0
        lax.fori_loop(0, CHUNK, zr, 0)
        for h in range(2):
            for ii in range(8):
                srows[X][h, pl.ds(ii * 16, 16)] = jnp.zeros((16,), jnp.int32)

    for l in range(1, LAYERS + 1):
        def zinit(i, _):
            tmpv[i, pl.ds(0, 16)] = jnp.zeros((16,), jnp.float32)
            tmpv[i, pl.ds(16, 16)] = jnp.zeros((16,), jnp.float32)
            return 0
        lax.fori_loop(0, RCHUNK, zinit, 0)

        def zbody(k, _):
            r0 = (s + k * NS) * RCHUNK
            pltpu.sync_copy(tmpv, acc_sh.at[pl.ds(r0, RCHUNK)])
            return 0
        lax.fori_loop(0, nrchunks_tile, zbody, 0)
        plsc.subcore_barrier()

        if l == 1:
            src_ref = all0_cs
            src_base = c * N
        else:
            src_ref = layers_hbm
            src_base = ((l - 2) * NC + c) * N

        def offs(X):
            for h in range(2):
                for ii in range(8):
                    sl = pl.ds(ii * 16, 16)
                    cols[X][h, sl] = cols[X][h, sl] + src_base

        def issue_gather(X):
            for h in range(2):
                pltpu.async_copy(src_ref.at[cols[X].at[h]], rowsd[X].at[h],
                                 semG[X])

        def drain_gather(X):
            for h in range(2):
                pltpu.make_async_copy(src_ref.at[cols[X].at[h]],
                                      rowsd[X].at[h], semG[X]).wait()

        def issue_scatter(X):
            for h in range(2):
                pltpu.async_copy(rowsd[X].at[h], acc_sh.at[srows[X].at[h]],
                                 semS[X], add=True)

        def drain_scatter(X):
            for h in range(2):
                pltpu.make_async_copy(rowsd[X].at[h],
                                      acc_sh.at[srows[X].at[h]], semS[X]).wait()

        for X in (1, 2):
            zero_set(X)
            issue_scatter(X)
        issue_idx(0, 0)
        issue_idx(1, 1)
        drain_idx(0)
        offs(0)
        issue_gather(0)

        def round_body(k, _):
            for u in range(SETS):
                j = k * SETS + u
                X, Y, P = u, (u + 1) % SETS, (u + 2) % SETS
                drain_idx(Y)
                offs(Y)
                drain_scatter(Y)
                issue_gather(Y)
                issue_idx(j + 2, P)
                drain_gather(X)
                mul_slot(X)
                snap_srow(X)
                issue_scatter(X)
            return 0
        lax.fori_loop(0, NSLOTS_TILE // SETS, round_body, 0)

        drain_scatter(1)
        drain_scatter(2)
        drain_gather(0)
        drain_idx(1)
        plsc.subcore_barrier()

        dst_base = ((l - 1) * NC + c) * N
        def fbody(k, _):
            r0 = (s + k * NS) * RCHUNK
            pltpu.sync_copy(acc_sh.at[pl.ds(r0, RCHUNK)], tmpv)
            pltpu.sync_copy(tmpv, layers_hbm.at[pl.ds(dst_base + r0, RCHUNK)])
            return 0
        lax.fori_loop(0, nrchunks_tile, fbody, 0)
        plsc.subcore_barrier()

    bidx = cols[0].at[0]
    boff = cols[1].at[0]
    gdst = rowsd[0].at[0]
    racc = rowsd[1].at[0]
    for idx_hbm, node_base, out_cs in (
        (user_inputs, 0, u_cs),
        (pos_groups, NUM_USER, p_cs),
        (neg_groups, NUM_USER, n_cs),
    ):
        for j in range(B // BCHUNK // NS):
            b0 = (s * (B // BCHUNK // NS) + j) * BCHUNK
            pltpu.sync_copy(idx_hbm.at[pl.ds(b0, BCHUNK)], bidx)
            for src_id in range(4):
                if src_id == 0:
                    base = node_base + c * N
                    src = all0_cs
                else:
                    base = node_base + ((src_id - 1) * NC + c) * N
                    src = layers_hbm
                for ii in range(8):
                    sl = pl.ds(ii * 16, 16)
                    boff[sl] = bidx[sl] + base
                pltpu.async_copy(src.at[boff], gdst, gsem).wait()

                def comb(r, _):
                    for g in range(2):
                        sl = pl.ds(g * 16, 16)
                        if src_id == 0:
                            racc[r, sl] = gdst[r, sl]
                        elif src_id == 3:
                            racc[r, sl] = (racc[r, sl] + gdst[r, sl]) * 0.25
                        else:
                            racc[r, sl] = racc[r, sl] + gdst[r, sl]
                    return 0
                lax.fori_loop(0, BCHUNK, comb, 0)
            pltpu.sync_copy(racc, out_cs.at[c, pl.ds(b0, BCHUNK)])

    w = s * NC + c
    for idx_hbm, table, out in (
        (user_inputs, user_table, u_ego),
        (pos_groups, group_table, p_ego),
        (neg_groups, group_table, n_ego),
    ):
        for h in range(BCHUNK // EGO_CHUNK):
            b0 = w * BCHUNK + h * EGO_CHUNK
            pltpu.sync_copy(idx_hbm.at[pl.ds(b0, EGO_CHUNK)], ebidx)
            pltpu.async_copy(table.at[ebidx], egorows, gsem).wait()
            pltpu.sync_copy(egorows, out.at[pl.ds(b0, EGO_CHUNK)])


_mesh = plsc.VectorSubcoreMesh(core_axis_name="c", subcore_axis_name="s",
                               num_cores=NC, num_subcores=NS)

_sc_call = pl.kernel(
    _sc_body,
    out_type=(
        jax.ShapeDtypeStruct((NC, B, HD), jnp.float32),
        jax.ShapeDtypeStruct((NC, B, HD), jnp.float32),
        jax.ShapeDtypeStruct((NC, B, HD), jnp.float32),
        jax.ShapeDtypeStruct((B, D), jnp.float32),
        jax.ShapeDtypeStruct((B, D), jnp.float32),
        jax.ShapeDtypeStruct((B, D), jnp.float32),
        jax.ShapeDtypeStruct((LAYERS * NC * N, HD), jnp.float32),
    ),
    mesh=_mesh,
    scratch_types=(
        [pltpu.VMEM((2, CHUNK), jnp.int32) for _ in range(3 * SETS)] +
        [pltpu.VMEM((SLOT,), jnp.float32) for _ in range(SETS)] +
        [pltpu.VMEM((2, CHUNK, HD), jnp.float32) for _ in range(SETS)] +
        [
            pltpu.VMEM((RCHUNK, HD), jnp.float32),
            pltpu.VMEM((EGO_CHUNK,), jnp.int32),
            pltpu.VMEM((EGO_CHUNK, D), jnp.float32),
            pltpu.VMEM_SHARED((N, HD), jnp.float32),
        ] +
        [pltpu.SemaphoreType.DMA for _ in range(3 * SETS + 1)]
    ),
    compiler_params=pltpu.CompilerParams(use_tc_tiling_on_sc=False),
)


def _pad_edges(a):
    a2 = a.reshape(NS, TILE_EDGES)
    a2 = jnp.pad(a2, ((0, 0), (0, TILE_SPAN - TILE_EDGES)))
    return jnp.pad(a2.reshape(-1), (0, E_PAD - NS * TILE_SPAN))


@jax.jit
def kernel(user_table, group_table, adj_val, adj_row, adj_col,
           user_inputs, pos_groups, neg_groups):
    all0 = jnp.concatenate([user_table, group_table], axis=0)
    all0_cs = all0.reshape(N, NC, HD).transpose(1, 0, 2).reshape(NC * N, HD)
    adj_valp = _pad_edges(adj_val)
    adj_rowp2 = _pad_edges(adj_row).reshape(E_PAD // CHUNK, CHUNK)
    adj_colp2 = _pad_edges(adj_col).reshape(E_PAD // CHUNK, CHUNK)
    u_cs, p_cs, n_cs, u_ego, p_ego, n_ego, _ = _sc_call(
        all0_cs, user_table, group_table, adj_valp, adj_rowp2, adj_colp2,
        user_inputs, pos_groups, neg_groups)
    user_embeds = jnp.concatenate([u_cs[0], u_cs[1]], axis=1)
    pos_embeds = jnp.concatenate([p_cs[0], p_cs[1]], axis=1)
    neg_embeds = jnp.concatenate([n_cs[0], n_cs[1]], axis=1)
    return (user_embeds, pos_embeds, neg_embeds, u_ego, p_ego, n_ego)


# speedup vs baseline: 1.2508x; 1.2508x over previous
import jax
import jax.numpy as jnp
from jax import lax
from jax.experimental import pallas as pl
from jax.experimental.pallas import tpu as pltpu
from jax.experimental.pallas import tpu_sc as plsc

NUM_USER = 30000
NUM_GROUP = 20000
N = NUM_USER + NUM_GROUP
E = 800000
D = 64
HD = D // 2
LAYERS = 3
B = 4096

NC = 2
NS = 16
CHUNK = 128
SLOT = 256
SETS = 3
TILE_EDGES = E // NS
NSLOTS_TILE = 201
TILE_SPAN = NSLOTS_TILE * SLOT
E_PAD = NS * TILE_SPAN + 2 * SLOT
RCHUNK = 40
NRCHUNKS = N // RCHUNK
EGO_CHUNK = 16
BCHUNK = 128


def _sc_body(all0_cs, user_table, group_table, adj_valp, adj_rowp2, adj_colp2,
             user_inputs, pos_groups, neg_groups,
             u_cs, p_cs, n_cs, u_ego, p_ego, n_ego, layers_hbm,
             col0, col1, col2, row0, row1, row2, srow0, srow1, srow2,
             val0, val1, val2, rows0, rows1, rows2,
             tmpv, ebidx, egorows, acc_sh,
             semI0, semI1, semI2, semG0, semG1, semG2,
             semS0, semS1, semS2, gsem):
    c = lax.axis_index("c")
    s = lax.axis_index("s")

    cols = [col0, col1, col2]
    rowsb = [row0, row1, row2]
    srows = [srow0, srow1, srow2]
    vals = [val0, val1, val2]
    rowsd = [rows0, rows1, rows2]
    semI = [semI0, semI1, semI2]
    semG = [semG0, semG1, semG2]
    semS = [semS0, semS1, semS2]

    nrchunks_tile = (NRCHUNKS - s + NS - 1) // NS
    base_e = s * TILE_SPAN
    base_ch = s * (TILE_SPAN // CHUNK)

    def issue_idx(j, X):
        e0 = base_e + j * SLOT
        ch0 = base_ch + 2 * j
        pltpu.async_copy(adj_colp2.at[pl.ds(ch0, 2)], cols[X], semI[X])
        pltpu.async_copy(adj_rowp2.at[pl.ds(ch0, 2)], rowsb[X], semI[X])
        pltpu.async_copy(adj_valp.at[pl.ds(e0, SLOT)], vals[X], semI[X])

    def drain_idx(X):
        pltpu.make_async_copy(adj_colp2.at[pl.ds(0, 2)], cols[X], semI[X]).wait()
        pltpu.make_async_copy(adj_rowp2.at[pl.ds(0, 2)], rowsb[X], semI[X]).wait()
        pltpu.make_async_copy(adj_valp.at[pl.ds(0, SLOT)], vals[X], semI[X]).wait()

    def snap_srow(X):
        for h in range(2):
            for ii in range(8):
                sl = pl.ds(ii * 16, 16)
                srows[X][h, sl] = rowsb[X][h, sl]

    def mul_slot(X):
        for h in range(2):
            for g in range(8):
                v0 = h * 128 + g * 16
                vv = vals[X][pl.ds(v0, 16)]
                for j in range(16):
                    vj = vv[j]
                    er = g * 16 + j
                    rowsd[X][h, er, pl.ds(0, 16)] = \
                        rowsd[X][h, er, pl.ds(0, 16)] * vj
                    rowsd[X][h, er, pl.ds(16, 16)] = \
                        rowsd[X][h, er, pl.ds(16, 16)] * vj

    def zero_set(X):
        def zr(r, _):
            for h in range(2):
                rowsd[X][h, r, pl.ds(0, 16)] = jnp.zeros((16,), jnp.float32)
                rowsd[X][h, r, pl.ds(16, 16)] = jnp.zeros((16,), jnp.float32)
            return 0
        lax.fori_loop(0, CHUNK, zr, 0)
        for h in range(2):
            for ii in range(8):
                srows[X][h, pl.ds(ii * 16, 16)] = jnp.zeros((16,), jnp.int32)

    for l in range(1, LAYERS + 1):
        def zinit(i, _):
            tmpv[i, pl.ds(0, 16)] = jnp.zeros((16,), jnp.float32)
            tmpv[i, pl.ds(16, 16)] = jnp.zeros((16,), jnp.float32)
            return 0
        lax.fori_loop(0, RCHUNK, zinit, 0)

        def zbody(k, _):
            r0 = (s + k * NS) * RCHUNK
            pltpu.sync_copy(tmpv, acc_sh.at[pl.ds(r0, RCHUNK)])
            return 0
        lax.fori_loop(0, nrchunks_tile, zbody, 0)
        plsc.subcore_barrier()

        if l == 1:
            src_ref = all0_cs
            src_base = c * N
        else:
            src_ref = layers_hbm
            src_base = ((l - 2) * NC + c) * N

        def offs(X):
            for h in range(2):
                for ii in range(8):
                    sl = pl.ds(ii * 16, 16)
                    cols[X][h, sl] = cols[X][h, sl] + src_base

        def issue_gather(X):
            for h in range(2):
                pltpu.async_copy(src_ref.at[cols[X].at[h]], rowsd[X].at[h],
                                 semG[X])

        def drain_gather(X):
            for h in range(2):
                pltpu.make_async_copy(src_ref.at[cols[X].at[h]],
                                      rowsd[X].at[h], semG[X]).wait()

        def issue_scatter(X):
            for h in range(2):
                pltpu.async_copy(rowsd[X].at[h], acc_sh.at[srows[X].at[h]],
                                 semS[X], add=True)

        def drain_scatter(X):
            for h in range(2):
                pltpu.make_async_copy(rowsd[X].at[h],
                                      acc_sh.at[srows[X].at[h]], semS[X]).wait()

        for X in (1, 2):
            zero_set(X)
            issue_scatter(X)
        issue_idx(0, 0)
        issue_idx(1, 1)
        drain_idx(0)
        offs(0)
        issue_gather(0)

        def round_body(k, _):
            for u in range(SETS):
                j = k * SETS + u
                X, Y, P = u, (u + 1) % SETS, (u + 2) % SETS
                drain_idx(Y)
                offs(Y)
                drain_scatter(Y)
                issue_gather(Y)
                issue_idx(j + 2, P)
                drain_gather(X)
                mul_slot(X)
                snap_srow(X)
                issue_scatter(X)
            return 0
        lax.fori_loop(0, NSLOTS_TILE // SETS, round_body, 0)

        drain_scatter(1)
        drain_scatter(2)
        drain_gather(0)
        drain_idx(1)
        plsc.subcore_barrier()

        dst_base = ((l - 1) * NC + c) * N
        def fbody(k, _):
            r0 = (s + k * NS) * RCHUNK
            pltpu.sync_copy(acc_sh.at[pl.ds(r0, RCHUNK)], tmpv)
            pltpu.sync_copy(tmpv, layers_hbm.at[pl.ds(dst_base + r0, RCHUNK)])
            return 0
        lax.fori_loop(0, nrchunks_tile, fbody, 0)
        plsc.subcore_barrier()

    bidx = cols[0]
    boff = cols[1]
    for idx_hbm, node_base, out_cs in (
        (user_inputs, 0, u_cs),
        (pos_groups, NUM_USER, p_cs),
        (neg_groups, NUM_USER, n_cs),
    ):
        for j in range(B // BCHUNK // NS):
            b0 = (s * (B // BCHUNK // NS) + j) * BCHUNK
            pltpu.sync_copy(idx_hbm.at[pl.ds(b0, BCHUNK)], bidx.at[0])
            for src_id in range(4):
                if src_id == 0:
                    base = node_base + c * N
                    src = all0_cs
                else:
                    base = node_base + ((src_id - 1) * NC + c) * N
                    src = layers_hbm
                for ii in range(8):
                    sl = pl.ds(ii * 16, 16)
                    boff[0, sl] = bidx[0, sl] + base
                pltpu.async_copy(src.at[boff.at[0]], rowsd[0].at[0],
                                 gsem).wait()

                def comb(r, _):
                    for g in range(2):
                        sl = pl.ds(g * 16, 16)
                        if src_id == 0:
                            rowsd[1][0, r, sl] = rowsd[0][0, r, sl]
                        elif src_id == 3:
                            rowsd[1][0, r, sl] = \
                                (rowsd[1][0, r, sl] + rowsd[0][0, r, sl]) * 0.25
                        else:
                            rowsd[1][0, r, sl] = \
                                rowsd[1][0, r, sl] + rowsd[0][0, r, sl]
                    return 0
                lax.fori_loop(0, BCHUNK, comb, 0)
            pltpu.sync_copy(rowsd[1].at[0], out_cs.at[c, pl.ds(b0, BCHUNK)])

    w = s * NC + c
    for idx_hbm, table, out in (
        (user_inputs, user_table, u_ego),
        (pos_groups, group_table, p_ego),
        (neg_groups, group_table, n_ego),
    ):
        for h in range(BCHUNK // EGO_CHUNK):
            b0 = w * BCHUNK + h * EGO_CHUNK
            pltpu.sync_copy(idx_hbm.at[pl.ds(b0, EGO_CHUNK)], ebidx)
            pltpu.async_copy(table.at[ebidx], egorows, gsem).wait()
            pltpu.sync_copy(egorows, out.at[pl.ds(b0, EGO_CHUNK)])


_mesh = plsc.VectorSubcoreMesh(core_axis_name="c", subcore_axis_name="s",
                               num_cores=NC, num_subcores=NS)

_sc_call = pl.kernel(
    _sc_body,
    out_type=(
        jax.ShapeDtypeStruct((NC, B, HD), jnp.float32),
        jax.ShapeDtypeStruct((NC, B, HD), jnp.float32),
        jax.ShapeDtypeStruct((NC, B, HD), jnp.float32),
        jax.ShapeDtypeStruct((B, D), jnp.float32),
        jax.ShapeDtypeStruct((B, D), jnp.float32),
        jax.ShapeDtypeStruct((B, D), jnp.float32),
        jax.ShapeDtypeStruct((LAYERS * NC * N, HD), jnp.float32),
    ),
    mesh=_mesh,
    scratch_types=(
        [pltpu.VMEM((2, CHUNK), jnp.int32) for _ in range(3 * SETS)] +
        [pltpu.VMEM((SLOT,), jnp.float32) for _ in range(SETS)] +
        [pltpu.VMEM((2, CHUNK, HD), jnp.float32) for _ in range(SETS)] +
        [
            pltpu.VMEM((RCHUNK, HD), jnp.float32),
            pltpu.VMEM((EGO_CHUNK,), jnp.int32),
            pltpu.VMEM((EGO_CHUNK, D), jnp.float32),
            pltpu.VMEM_SHARED((N, HD), jnp.float32),
        ] +
        [pltpu.SemaphoreType.DMA for _ in range(3 * SETS + 1)]
    ),
    compiler_params=pltpu.CompilerParams(use_tc_tiling_on_sc=False),
)


def _pad_edges(a):
    a2 = a.reshape(NS, TILE_EDGES)
    a2 = jnp.pad(a2, ((0, 0), (0, TILE_SPAN - TILE_EDGES)))
    return jnp.pad(a2.reshape(-1), (0, E_PAD - NS * TILE_SPAN))


@jax.jit
def kernel(user_table, group_table, adj_val, adj_row, adj_col,
           user_inputs, pos_groups, neg_groups):
    all0 = jnp.concatenate([user_table, group_table], axis=0)
    all0_cs = all0.reshape(N, NC, HD).transpose(1, 0, 2).reshape(NC * N, HD)
    adj_valp = _pad_edges(adj_val)
    adj_rowp2 = _pad_edges(adj_row).reshape(E_PAD // CHUNK, CHUNK)
    adj_colp2 = _pad_edges(adj_col).reshape(E_PAD // CHUNK, CHUNK)
    u_cs, p_cs, n_cs, u_ego, p_ego, n_ego, _ = _sc_call(
        all0_cs, user_table, group_table, adj_valp, adj_rowp2, adj_colp2,
        user_inputs, pos_groups, neg_groups)
    user_embeds = jnp.concatenate([u_cs[0], u_cs[1]], axis=1)
    pos_embeds = jnp.concatenate([p_cs[0], p_cs[1]], axis=1)
    neg_embeds = jnp.concatenate([n_cs[0], n_cs[1]], axis=1)
    return (user_embeds, pos_embeds, neg_embeds, u_ego, p_ego, n_ego)


# speedup vs baseline: 1.8660x; 1.4918x over previous
import jax
import jax.numpy as jnp
from jax import lax
from jax.experimental import pallas as pl
from jax.experimental.pallas import tpu as pltpu
from jax.experimental.pallas import tpu_sc as plsc

NUM_USER = 30000
NUM_GROUP = 20000
N = NUM_USER + NUM_GROUP
E = 800000
D = 64
HD = D // 2
LAYERS = 3
B = 4096

NC = 2
NS = 16
CHUNK = 128
SETS = 4
TILE_EDGES = E // NS
NCHUNKS_TILE = 392
TILE_SPAN = NCHUNKS_TILE * CHUNK
E_PAD = NS * TILE_SPAN + 2 * CHUNK
RCHUNK = 100
NRCHUNKS = N // RCHUNK
EGO_CHUNK = 64
BCHUNK = 128


def _mul16(rows_ref, cvr_ref):
    for i in range(8):
        e0 = i * 16
        vv = plsc.bitcast(cvr_ref[0, 1, pl.ds(e0, 16)], jnp.float32)
        for j in range(16):
            v = vv[j]
            e = e0 + j
            rows_ref[e, pl.ds(0, 16)] = rows_ref[e, pl.ds(0, 16)] * v
            rows_ref[e, pl.ds(16, 16)] = rows_ref[e, pl.ds(16, 16)] * v


def _add_base(dst_ref, src_ref, base):
    for i in range(8):
        s = pl.ds(i * 16, 16)
        dst_ref[s] = src_ref[s] + base


def _sc_body(all0_cs, user_table, group_table, edge_cvr,
             user_inputs, pos_groups, neg_groups,
             u_cs, p_cs, n_cs, u_ego, p_ego, n_ego, layers_hbm,
             cvr0, cvr1, cvr2, cvr3, rows0, rows1, rows2, rows3,
             tmpv, rowsum, bidx, boff, ebidx, egorows, acc_sh,
             semI0, semI1, semI2, semI3, semG0, semG1, semG2, semG3,
             semS0, semS1, semS2, semS3, gsem):
    c = lax.axis_index("c")
    s = lax.axis_index("s")

    cvrs = [cvr0, cvr1, cvr2, cvr3]
    rowsd = [rows0, rows1, rows2, rows3]
    semI = [semI0, semI1, semI2, semI3]
    semG = [semG0, semG1, semG2, semG3]
    semS = [semS0, semS1, semS2, semS3]

    nrchunks_tile = (NRCHUNKS - s + NS - 1) // NS
    base_ch = s * NCHUNKS_TILE

    def issue_idx(i, X):
        pltpu.async_copy(edge_cvr.at[pl.ds(base_ch + i, 1)], cvrs[X], semI[X])

    def drain_idx(X):
        pltpu.make_async_copy(edge_cvr.at[pl.ds(0, 1)], cvrs[X],
                              semI[X]).wait()

    def zero_rows(X):
        def zr(r, _):
            rowsd[X][r, pl.ds(0, 16)] = jnp.zeros((16,), jnp.float32)
            rowsd[X][r, pl.ds(16, 16)] = jnp.zeros((16,), jnp.float32)
            return 0
        lax.fori_loop(0, CHUNK, zr, 0)
        for ii in range(8):
            cvrs[X][0, 2, pl.ds(ii * 16, 16)] = jnp.zeros((16,), jnp.int32)

    for l in range(1, LAYERS + 1):
        def zinit(i, _):
            tmpv[i, pl.ds(0, 16)] = jnp.zeros((16,), jnp.float32)
            tmpv[i, pl.ds(16, 16)] = jnp.zeros((16,), jnp.float32)
            return 0
        lax.fori_loop(0, RCHUNK, zinit, 0)

        def zbody(k, _):
            r0 = (s + k * NS) * RCHUNK
            pltpu.sync_copy(tmpv, acc_sh.at[pl.ds(r0, RCHUNK)])
            return 0
        lax.fori_loop(0, nrchunks_tile, zbody, 0)
        plsc.subcore_barrier()

        if l == 1:
            src_ref = all0_cs
            src_base = c * N
        else:
            src_ref = layers_hbm
            src_base = ((l - 2) * NC + c) * N

        def offs(X):
            for ii in range(8):
                sl = pl.ds(ii * 16, 16)
                cvrs[X][0, 0, sl] = cvrs[X][0, 0, sl] + src_base

        def issue_gather(X):
            pltpu.async_copy(src_ref.at[cvrs[X].at[0, 0]], rowsd[X], semG[X])

        def drain_gather(X):
            pltpu.make_async_copy(src_ref.at[cvrs[X].at[0, 0]], rowsd[X],
                                  semG[X]).wait()

        def issue_scatter(X):
            pltpu.async_copy(rowsd[X], acc_sh.at[cvrs[X].at[0, 2]], semS[X],
                             add=True)

        def drain_scatter(X):
            pltpu.make_async_copy(rowsd[X], acc_sh.at[cvrs[X].at[0, 2]],
                                  semS[X]).wait()

        for X in (2, 3):
            zero_rows(X)
            issue_scatter(X)
        issue_idx(0, 0)
        issue_idx(1, 1)
        drain_idx(0)
        offs(0)
        issue_gather(0)

        def round_body(k, _):
            for u in range(SETS):
                i = k * SETS + u
                X, Y, P = u, (u + 1) % SETS, (u + 2) % SETS
                drain_idx(Y)
                offs(Y)
                drain_scatter(P)
                issue_gather(Y)
                issue_idx(i + 2, P)
                drain_gather(X)
                _mul16(rowsd[X], cvrs[X])
                issue_scatter(X)
            return 0
        lax.fori_loop(0, NCHUNKS_TILE // SETS, round_body, 0)

        drain_scatter(2)
        drain_scatter(3)
        drain_gather(0)
        drain_idx(1)
        plsc.subcore_barrier()

        dst_base = ((l - 1) * NC + c) * N
        def fbody(k, _):
            r0 = (s + k * NS) * RCHUNK
            pltpu.sync_copy(acc_sh.at[pl.ds(r0, RCHUNK)], tmpv)
            pltpu.sync_copy(tmpv, layers_hbm.at[pl.ds(dst_base + r0, RCHUNK)])
            return 0
        lax.fori_loop(0, nrchunks_tile, fbody, 0)
        plsc.subcore_barrier()

    for idx_hbm, node_base, out_cs in (
        (user_inputs, 0, u_cs),
        (pos_groups, NUM_USER, p_cs),
        (neg_groups, NUM_USER, n_cs),
    ):
        for j in range(B // BCHUNK // NS):
            b0 = (s * (B // BCHUNK // NS) + j) * BCHUNK
            pltpu.sync_copy(idx_hbm.at[pl.ds(b0, BCHUNK)], bidx)
            for src_id in range(4):
                if src_id == 0:
                    base = node_base + c * N
                    src = all0_cs
                else:
                    base = node_base + ((src_id - 1) * NC + c) * N
                    src = layers_hbm
                _add_base(boff, bidx, base)
                pltpu.async_copy(src.at[boff], rows0, gsem).wait()

                def comb(r, _):
                    for g in range(2):
                        sl = pl.ds(g * 16, 16)
                        if src_id == 0:
                            rowsum[r, sl] = rows0[r, sl]
                        elif src_id == 3:
                            rowsum[r, sl] = (rowsum[r, sl] + rows0[r, sl]) * 0.25
                        else:
                            rowsum[r, sl] = rowsum[r, sl] + rows0[r, sl]
                    return 0
                lax.fori_loop(0, BCHUNK, comb, 0)
            pltpu.sync_copy(rowsum, out_cs.at[c, pl.ds(b0, BCHUNK)])

    w = s * NC + c
    for idx_hbm, table, out in (
        (user_inputs, user_table, u_ego),
        (pos_groups, group_table, p_ego),
        (neg_groups, group_table, n_ego),
    ):
        for h in range(BCHUNK // EGO_CHUNK):
            b0 = w * BCHUNK + h * EGO_CHUNK
            pltpu.sync_copy(idx_hbm.at[pl.ds(b0, EGO_CHUNK)], ebidx)
            pltpu.async_copy(table.at[ebidx], egorows, gsem).wait()
            pltpu.sync_copy(egorows, out.at[pl.ds(b0, EGO_CHUNK)])


_mesh = plsc.VectorSubcoreMesh(core_axis_name="c", subcore_axis_name="s",
                               num_cores=NC, num_subcores=NS)

_sc_call = pl.kernel(
    _sc_body,
    out_type=(
        jax.ShapeDtypeStruct((NC, B, HD), jnp.float32),
        jax.ShapeDtypeStruct((NC, B, HD), jnp.float32),
        jax.ShapeDtypeStruct((NC, B, HD), jnp.float32),
        jax.ShapeDtypeStruct((B, D), jnp.float32),
        jax.ShapeDtypeStruct((B, D), jnp.float32),
        jax.ShapeDtypeStruct((B, D), jnp.float32),
        jax.ShapeDtypeStruct((LAYERS * NC * N, HD), jnp.float32),
    ),
    mesh=_mesh,
    scratch_types=(
        [pltpu.VMEM((1, 3, CHUNK), jnp.int32) for _ in range(SETS)] +
        [pltpu.VMEM((CHUNK, HD), jnp.float32) for _ in range(SETS)] +
        [
            pltpu.VMEM((RCHUNK, HD), jnp.float32),
            pltpu.VMEM((BCHUNK, HD), jnp.float32),
            pltpu.VMEM((BCHUNK,), jnp.int32),
            pltpu.VMEM((BCHUNK,), jnp.int32),
            pltpu.VMEM((EGO_CHUNK,), jnp.int32),
            pltpu.VMEM((EGO_CHUNK, D), jnp.float32),
            pltpu.VMEM_SHARED((N, HD), jnp.float32),
        ] +
        [pltpu.SemaphoreType.DMA for _ in range(3 * SETS + 1)]
    ),
    compiler_params=pltpu.CompilerParams(use_tc_tiling_on_sc=False,
                                         needs_layout_passes=False),
)


def _pad_edges(a):
    a2 = a.reshape(NS, TILE_EDGES)
    a2 = jnp.pad(a2, ((0, 0), (0, TILE_SPAN - TILE_EDGES)))
    return jnp.pad(a2.reshape(-1), (0, E_PAD - NS * TILE_SPAN))


@jax.jit
def kernel(user_table, group_table, adj_val, adj_row, adj_col,
           user_inputs, pos_groups, neg_groups):
    all0 = jnp.concatenate([user_table, group_table], axis=0)
    all0_cs = all0.reshape(N, NC, HD).transpose(1, 0, 2).reshape(NC * N, HD)
    colp = _pad_edges(adj_col).reshape(-1, CHUNK)
    valp = lax.bitcast_convert_type(_pad_edges(adj_val),
                                    jnp.int32).reshape(-1, CHUNK)
    rowp = _pad_edges(adj_row).reshape(-1, CHUNK)
    edge_cvr = jnp.stack([colp, valp, rowp], axis=1)
    u_cs, p_cs, n_cs, u_ego, p_ego, n_ego, _ = _sc_call(
        all0_cs, user_table, group_table, edge_cvr,
        user_inputs, pos_groups, neg_groups)
    user_embeds = jnp.concatenate([u_cs[0], u_cs[1]], axis=1)
    pos_embeds = jnp.concatenate([p_cs[0], p_cs[1]], axis=1)
    neg_embeds = jnp.concatenate([n_cs[0], n_cs[1]], axis=1)
    return (user_embeds, pos_embeds, neg_embeds, u_ego, p_ego, n_ego)


# speedup vs baseline: 1.8763x; 1.0055x over previous
import jax
import jax.numpy as jnp
from jax import lax
from jax.experimental import pallas as pl
from jax.experimental.pallas import tpu as pltpu
from jax.experimental.pallas import tpu_sc as plsc

NUM_USER = 30000
NUM_GROUP = 20000
N = NUM_USER + NUM_GROUP
E = 800000
D = 64
HD = D // 2
LAYERS = 3
B = 4096

NC = 2
NS = 16
CHUNK = 128
SETS = 4
TILE_EDGES = E // NS
NCHUNKS_TILE = 392
TILE_SPAN = NCHUNKS_TILE * CHUNK
E_PAD = NS * TILE_SPAN + 2 * CHUNK
RCHUNK = 100
NRCHUNKS = N // RCHUNK
EGO_CHUNK = 64
BCHUNK = 128


def _mul16(rows_ref, cvr_ref):
    for i in range(8):
        e0 = i * 16
        vv = plsc.bitcast(cvr_ref[0, 1, pl.ds(e0, 16)], jnp.float32)
        for j in range(16):
            v = vv[j]
            e = e0 + j
            rows_ref[e, pl.ds(0, 16)] = rows_ref[e, pl.ds(0, 16)] * v
            rows_ref[e, pl.ds(16, 16)] = rows_ref[e, pl.ds(16, 16)] * v


def _add_base(dst_ref, src_ref, base):
    for i in range(8):
        s = pl.ds(i * 16, 16)
        dst_ref[s] = src_ref[s] + base


def _sc_body(all0_cs, user_table, group_table, edge_cvr,
             user_inputs, pos_groups, neg_groups,
             u_cs, p_cs, n_cs, u_ego, p_ego, n_ego, layers_hbm,
             cvr0, cvr1, cvr2, cvr3, rows0, rows1, rows2, rows3,
             tmpv, rowsum, bidx, boff, ebidx, egorows, acc_sh,
             semI0, semI1, semI2, semI3, semG0, semG1, semG2, semG3,
             semS0, semS1, semS2, semS3, gsem):
    c = lax.axis_index("c")
    s = lax.axis_index("s")

    cvrs = [cvr0, cvr1, cvr2, cvr3]
    rowsd = [rows0, rows1, rows2, rows3]
    semI = [semI0, semI1, semI2, semI3]
    semG = [semG0, semG1, semG2, semG3]
    semS = [semS0, semS1, semS2, semS3]

    nrchunks_tile = (NRCHUNKS - s + NS - 1) // NS
    base_ch = s * NCHUNKS_TILE

    def issue_idx(i, X):
        pltpu.async_copy(edge_cvr.at[pl.ds(base_ch + i, 1)], cvrs[X], semI[X])

    def drain_idx(X):
        pltpu.make_async_copy(edge_cvr.at[pl.ds(0, 1)], cvrs[X],
                              semI[X]).wait()

    def zero_rows(X):
        def zr(r, _):
            rowsd[X][r, pl.ds(0, 16)] = jnp.zeros((16,), jnp.float32)
            rowsd[X][r, pl.ds(16, 16)] = jnp.zeros((16,), jnp.float32)
            return 0
        lax.fori_loop(0, CHUNK, zr, 0)
        for ii in range(8):
            cvrs[X][0, 2, pl.ds(ii * 16, 16)] = jnp.zeros((16,), jnp.int32)

    for l in range(1, LAYERS + 1):
        def zinit(i, _):
            tmpv[i, pl.ds(0, 16)] = jnp.zeros((16,), jnp.float32)
            tmpv[i, pl.ds(16, 16)] = jnp.zeros((16,), jnp.float32)
            return 0
        lax.fori_loop(0, RCHUNK, zinit, 0)

        def zbody(k, _):
            r0 = (s + k * NS) * RCHUNK
            pltpu.sync_copy(tmpv, acc_sh.at[pl.ds(r0, RCHUNK)])
            return 0
        lax.fori_loop(0, nrchunks_tile, zbody, 0)
        plsc.subcore_barrier()

        if l == 1:
            src_view = all0_cs.at[pl.ds(c * N, N)]
        else:
            src_view = layers_hbm.at[pl.ds(((l - 2) * NC + c) * N, N)]

        def issue_gather(X):
            pltpu.async_copy(src_view.at[cvrs[X].at[0, 0]], rowsd[X], semG[X])

        def drain_gather(X):
            pltpu.make_async_copy(src_view.at[cvrs[X].at[0, 0]], rowsd[X],
                                  semG[X]).wait()

        def issue_scatter(X):
            pltpu.async_copy(rowsd[X], acc_sh.at[cvrs[X].at[0, 2]], semS[X],
                             add=True)

        def drain_scatter(X):
            pltpu.make_async_copy(rowsd[X], acc_sh.at[cvrs[X].at[0, 2]],
                                  semS[X]).wait()

        for X in (2, 3):
            zero_rows(X)
            issue_scatter(X)
        issue_idx(0, 0)
        issue_idx(1, 1)
        drain_idx(0)
        issue_gather(0)

        def round_body(k, _):
            for u in range(SETS):
                i = k * SETS + u
                X, Y, P = u, (u + 1) % SETS, (u + 2) % SETS
                drain_idx(Y)
                drain_scatter(P)
                issue_gather(Y)
                issue_idx(i + 2, P)
                drain_gather(X)
                _mul16(rowsd[X], cvrs[X])
                issue_scatter(X)
            return 0
        lax.fori_loop(0, NCHUNKS_TILE // SETS, round_body, 0)

        drain_scatter(2)
        drain_scatter(3)
        drain_gather(0)
        drain_idx(1)
        plsc.subcore_barrier()

        dst_base = ((l - 1) * NC + c) * N
        def fbody(k, _):
            r0 = (s + k * NS) * RCHUNK
            pltpu.sync_copy(acc_sh.at[pl.ds(r0, RCHUNK)], tmpv)
            pltpu.sync_copy(tmpv, layers_hbm.at[pl.ds(dst_base + r0, RCHUNK)])
            return 0
        lax.fori_loop(0, nrchunks_tile, fbody, 0)
        plsc.subcore_barrier()

    for idx_hbm, node_base, out_cs in (
        (user_inputs, 0, u_cs),
        (pos_groups, NUM_USER, p_cs),
        (neg_groups, NUM_USER, n_cs),
    ):
        for j in range(B // BCHUNK // NS):
            b0 = (s * (B // BCHUNK // NS) + j) * BCHUNK
            pltpu.sync_copy(idx_hbm.at[pl.ds(b0, BCHUNK)], bidx)
            for src_id in range(4):
                if src_id == 0:
                    base = node_base + c * N
                    src = all0_cs
                else:
                    base = node_base + ((src_id - 1) * NC + c) * N
                    src = layers_hbm
                _add_base(boff, bidx, base)
                pltpu.async_copy(src.at[boff], rows0, gsem).wait()

                def comb(r, _):
                    for g in range(2):
                        sl = pl.ds(g * 16, 16)
                        if src_id == 0:
                            rowsum[r, sl] = rows0[r, sl]
                        elif src_id == 3:
                            rowsum[r, sl] = (rowsum[r, sl] + rows0[r, sl]) * 0.25
                        else:
                            rowsum[r, sl] = rowsum[r, sl] + rows0[r, sl]
                    return 0
                lax.fori_loop(0, BCHUNK, comb, 0)
            pltpu.sync_copy(rowsum, out_cs.at[c, pl.ds(b0, BCHUNK)])

    w = s * NC + c
    for idx_hbm, table, out in (
        (user_inputs, user_table, u_ego),
        (pos_groups, group_table, p_ego),
        (neg_groups, group_table, n_ego),
    ):
        for h in range(BCHUNK // EGO_CHUNK):
            b0 = w * BCHUNK + h * EGO_CHUNK
            pltpu.sync_copy(idx_hbm.at[pl.ds(b0, EGO_CHUNK)], ebidx)
            pltpu.async_copy(table.at[ebidx], egorows, gsem).wait()
            pltpu.sync_copy(egorows, out.at[pl.ds(b0, EGO_CHUNK)])


_mesh = plsc.VectorSubcoreMesh(core_axis_name="c", subcore_axis_name="s",
                               num_cores=NC, num_subcores=NS)

_sc_call = pl.kernel(
    _sc_body,
    out_type=(
        jax.ShapeDtypeStruct((NC, B, HD), jnp.float32),
        jax.ShapeDtypeStruct((NC, B, HD), jnp.float32),
        jax.ShapeDtypeStruct((NC, B, HD), jnp.float32),
        jax.ShapeDtypeStruct((B, D), jnp.float32),
        jax.ShapeDtypeStruct((B, D), jnp.float32),
        jax.ShapeDtypeStruct((B, D), jnp.float32),
        jax.ShapeDtypeStruct((LAYERS * NC * N, HD), jnp.float32),
    ),
    mesh=_mesh,
    scratch_types=(
        [pltpu.VMEM((1, 3, CHUNK), jnp.int32) for _ in range(SETS)] +
        [pltpu.VMEM((CHUNK, HD), jnp.float32) for _ in range(SETS)] +
        [
            pltpu.VMEM((RCHUNK, HD), jnp.float32),
            pltpu.VMEM((BCHUNK, HD), jnp.float32),
            pltpu.VMEM((BCHUNK,), jnp.int32),
            pltpu.VMEM((BCHUNK,), jnp.int32),
            pltpu.VMEM((EGO_CHUNK,), jnp.int32),
            pltpu.VMEM((EGO_CHUNK, D), jnp.float32),
            pltpu.VMEM_SHARED((N, HD), jnp.float32),
        ] +
        [pltpu.SemaphoreType.DMA for _ in range(3 * SETS + 1)]
    ),
    compiler_params=pltpu.CompilerParams(use_tc_tiling_on_sc=False,
                                         needs_layout_passes=False),
)


def _pad_edges(a):
    a2 = a.reshape(NS, TILE_EDGES)
    a2 = jnp.pad(a2, ((0, 0), (0, TILE_SPAN - TILE_EDGES)))
    return jnp.pad(a2.reshape(-1), (0, E_PAD - NS * TILE_SPAN))


@jax.jit
def kernel(user_table, group_table, adj_val, adj_row, adj_col,
           user_inputs, pos_groups, neg_groups):
    all0 = jnp.concatenate([user_table, group_table], axis=0)
    all0_cs = all0.reshape(N, NC, HD).transpose(1, 0, 2).reshape(NC * N, HD)
    colp = _pad_edges(adj_col).reshape(-1, CHUNK)
    valp = lax.bitcast_convert_type(_pad_edges(adj_val),
                                    jnp.int32).reshape(-1, CHUNK)
    rowp = _pad_edges(adj_row).reshape(-1, CHUNK)
    edge_cvr = jnp.stack([colp, valp, rowp], axis=1)
    u_cs, p_cs, n_cs, u_ego, p_ego, n_ego, _ = _sc_call(
        all0_cs, user_table, group_table, edge_cvr,
        user_inputs, pos_groups, neg_groups)
    user_embeds = jnp.concatenate([u_cs[0], u_cs[1]], axis=1)
    pos_embeds = jnp.concatenate([p_cs[0], p_cs[1]], axis=1)
    neg_embeds = jnp.concatenate([n_cs[0], n_cs[1]], axis=1)
    return (user_embeds, pos_embeds, neg_embeds, u_ego, p_ego, n_ego)
